# baseline (speedup 1.0000x reference)
import jax
import jax.numpy as jnp
from jax.experimental import pallas as pl
from jax.experimental.pallas import tpu as pltpu

_DC = 1024
_NB = 2048


def _pconv_block(x_ref, w_ref, o_ref):
    h = pl.program_id(2)

    @pl.when(h == 0)
    def _mm():
        o_ref[0, 0] = jax.lax.dot(
            w_ref[...].astype(jnp.bfloat16),
            x_ref[0, 0].astype(jnp.bfloat16),
            preferred_element_type=jnp.float32,
        )

    @pl.when(h == 1)
    def _copy():
        o_ref[0, 0] = x_ref[0, 0]


def kernel(x, W):
    b, dim, n = x.shape
    xv = x.reshape(b, 2, _DC, n)
    grid = (b, n // _NB, 2)
    out = pl.pallas_call(
        _pconv_block,
        grid=grid,
        in_specs=[
            pl.BlockSpec((1, 1, _DC, _NB), lambda i, j, h: (i, h, 0, j)),
            pl.BlockSpec((_DC, _DC), lambda i, j, h: (0, 0)),
        ],
        out_specs=pl.BlockSpec((1, 1, _DC, _NB), lambda i, j, h: (i, h, 0, j)),
        out_shape=jax.ShapeDtypeStruct((b, 2, _DC, n), x.dtype),
        compiler_params=pltpu.CompilerParams(
            dimension_semantics=("parallel", "parallel", "arbitrary"),
        ),
    )(xv, W)
    return out.reshape(b, dim, n)


# speedup vs baseline: 1.0939x; 1.0939x over previous
import jax
import jax.numpy as jnp
from jax.experimental import pallas as pl
from jax.experimental.pallas import tpu as pltpu

_DIM = 2048
_DC = 1024
_NB = 1024


def _pconv_block(x1_ref, x2_ref, w_ref, o_ref):
    o_ref[0, _DC:, :] = x2_ref[0]
    o_ref[0, :_DC, :] = jax.lax.dot(
        w_ref[...].astype(jnp.bfloat16),
        x1_ref[0].astype(jnp.bfloat16),
        preferred_element_type=jnp.float32,
    )


def kernel(x, W):
    b, dim, n = x.shape
    grid = (b, n // _NB)
    return pl.pallas_call(
        _pconv_block,
        grid=grid,
        in_specs=[
            pl.BlockSpec((1, _DC, _NB), lambda i, j: (i, 0, j)),
            pl.BlockSpec((1, _DC, _NB), lambda i, j: (i, 1, j)),
            pl.BlockSpec((_DC, _DC), lambda i, j: (0, 0)),
        ],
        out_specs=pl.BlockSpec((1, _DIM, _NB), lambda i, j: (i, 0, j)),
        out_shape=jax.ShapeDtypeStruct((b, dim, n), x.dtype),
        compiler_params=pltpu.CompilerParams(
            dimension_semantics=("parallel", "parallel"),
        ),
    )(x, x, W)
